# baseline (speedup 1.0000x reference)
import jax
import jax.numpy as jnp
from jax.experimental import pallas as pl

_EPS = 1e-05
_N_NODES = 100000
_N_FEATURES = 512
_BLOCK_ROWS = 5000


def _stfn_block(x_ref, w_ref, b_ref, o_ref):
    x = x_ref[...]
    mean = jnp.mean(x, axis=1, keepdims=True)
    xc = x - mean
    var = jnp.mean(xc * xc, axis=1, keepdims=True)
    inv = jax.lax.rsqrt(var + _EPS)
    o_ref[...] = (xc * inv) * w_ref[...] + b_ref[...]


def kernel(input, weight, bias):
    n, c = input.shape
    grid = (n // _BLOCK_ROWS,)
    return pl.pallas_call(
        _stfn_block,
        grid=grid,
        in_specs=[
            pl.BlockSpec((_BLOCK_ROWS, c), lambda i: (i, 0)),
            pl.BlockSpec((1, c), lambda i: (0, 0)),
            pl.BlockSpec((1, c), lambda i: (0, 0)),
        ],
        out_specs=pl.BlockSpec((_BLOCK_ROWS, c), lambda i: (i, 0)),
        out_shape=jax.ShapeDtypeStruct((n, c), input.dtype),
    )(input, weight.reshape(1, c), bias.reshape(1, c))
